# baseline (speedup 1.0000x reference)
import functools

import jax
import jax.numpy as jnp
from jax import lax
from jax.experimental import pallas as pl
from jax.experimental.pallas import tpu as pltpu
from jax.experimental.pallas import tpu_sc as plsc

N_NODES = 10000
N_EDGES = 320000
D = 128

NC = 2
NS = 16
NW = NC * NS

E_PER_W = N_EDGES // NW
CHUNK = 80
CHUNK_W = 128
N_CHUNKS = E_PER_W // CHUNK
NSLOT = 4
NRSLOT = 2

ZR = 24
WB_ROWS = 624
WB_TAIL = N_NODES - NS * WB_ROWS

MM_BLK = 2000


def _mm_body(x_ref, w_ref, o_ref):
    sup = jnp.dot(x_ref[...], w_ref[...], preferred_element_type=jnp.float32)
    sup = sup.reshape(MM_BLK, D // 32, 2, 16)
    sup = jnp.swapaxes(sup, 2, 3)
    o_ref[...] = sup.reshape(MM_BLK, D).astype(jnp.bfloat16)


def _tc_matmul(x, W):
    return pl.pallas_call(
        _mm_body,
        grid=(N_NODES // MM_BLK,),
        in_specs=[
            pl.BlockSpec((MM_BLK, D), lambda i: (i, 0)),
            pl.BlockSpec((D, D), lambda i: (0, 0)),
        ],
        out_specs=pl.BlockSpec((MM_BLK, D), lambda i: (i, 0)),
        out_shape=jax.ShapeDtypeStruct((N_NODES, D), jnp.bfloat16),
    )(x, W)


def _fin_body(p_ref, b_ref, a_ref, o_ref):
    y = p_ref[0] + p_ref[1] + b_ref[...]
    a = a_ref[0]
    o_ref[...] = jnp.where(y >= 0, y, a * y)


def _tc_finish(parts, b, prelu_a):
    return pl.pallas_call(
        _fin_body,
        grid=(5,),
        in_specs=[
            pl.BlockSpec((2, 2000, D), lambda i: (0, i, 0)),
            pl.BlockSpec((1, D), lambda i: (0, 0)),
            pl.BlockSpec(memory_space=pltpu.SMEM),
        ],
        out_specs=pl.BlockSpec((2000, D), lambda i: (i, 0)),
        out_shape=jax.ShapeDtypeStruct((N_NODES, D), jnp.float32),
    )(parts, b.reshape(1, D), prelu_a.reshape(1))


def _sc_body(support, edge, ew, out, acc,
             eb0, eb1, eb2, eb3, sb0, sb1, sb2, sb3, db0, db1, db2, db3,
             wb0, wb1, wb2, wb3, gb0, gb1, gb2, gb3, rw0, rw1, zbuf,
             p0, p1, p2, p3, g0, g1, g2, g3, s0, s1, s2, s3, zsem):
    eb = [eb0, eb1, eb2, eb3]
    sb = [sb0, sb1, sb2, sb3]
    db = [db0, db1, db2, db3]
    wb = [wb0, wb1, wb2, wb3]
    gb = [gb0, gb1, gb2, gb3]
    rw = [rw0, rw1]
    psem = [p0, p1, p2, p3]
    gsem = [g0, g1, g2, g3]
    ssem = [s0, s1, s2, s3]

    c = lax.axis_index("c")
    s = lax.axis_index("s")
    wid = s * NC + c

    base0 = wid * E_PER_W

    def _slices(m):
        base = base0 + m * CHUNK
        al = pl.multiple_of(jnp.minimum((base // 128) * 128,
                                        N_EDGES - 2 * CHUNK_W), 128)
        off = pl.multiple_of(base - al, 8)
        return (edge.at[pl.ds(0, 2), pl.ds(al, 2 * CHUNK_W)],
                ew.at[pl.ds(pl.multiple_of(base, 8), CHUNK)],
                off)

    def load_idx(m, sl):
        sedge, sew, _ = _slices(m)
        pltpu.async_copy(sedge, eb[sl], psem[sl])
        pltpu.async_copy(sew, wb[sl], psem[sl])

    def wait_idx(m, sl):
        sedge, sew, _ = _slices(m)
        pltpu.make_async_copy(sedge, eb[sl], psem[sl]).wait()
        pltpu.make_async_copy(sew, wb[sl], psem[sl]).wait()

    def extract_idx(m, sl):
        _, _, off = _slices(m)
        for j in range(CHUNK // 16):
            jsl = pl.ds(j * 16, 16)
            sb[sl][jsl] = eb[sl][1, pl.ds(off + j * 16, 16)]
            db[sl][jsl] = eb[sl][0, pl.ds(off + j * 16, 16)]

    def start_gather(m, sl):
        pltpu.async_copy(support.at[sb[sl]], gb[sl], gsem[sl])

    def wait_gather(m, sl):
        pltpu.make_async_copy(support.at[sb[sl]], gb[sl], gsem[sl]).wait()

    def start_scatter(sl, slr):
        pltpu.async_copy(rw[slr], acc.at[db[sl]], ssem[sl], add=True)

    def wait_scatter(sl, slr):
        pltpu.make_async_copy(rw[slr], acc.at[db[sl]], ssem[sl]).wait()

    def multiply(sl, slr):
        @pl.loop(0, CHUNK // 16)
        def _grp(g):
            wv = wb[sl][pl.ds(g * 16, 16)]
            for j in range(16):
                w = jnp.full((16,), wv[j], jnp.float32)
                e = g * 16 + j
                for q in range(D // 32):
                    v = plsc.bitcast(gb[sl][e, pl.ds(q * 16, 16)],
                                     jnp.bfloat16)
                    lo, hi = plsc.unpack(v, format=plsc.PackFormat.INTERLEAVED)
                    rw[slr][e, pl.ds(q * 32, 16)] = lo * w
                    rw[slr][e, pl.ds(q * 32 + 16, 16)] = hi * w

    def process(m, sl, slr, prep_gather, prep_idx, wait_sc):
        wait_gather(m, sl)
        if wait_sc:
            wait_scatter((sl + 2) % NSLOT, slr)
        multiply(sl, slr)
        start_scatter(sl, slr)
        if prep_gather:
            m2 = m + 2
            sl2 = (sl + 2) % NSLOT
            wait_idx(m2, sl2)
            extract_idx(m2, sl2)
            start_gather(m2, sl2)
        if prep_idx:
            load_idx(m + 3, (sl + 3) % NSLOT)

    load_idx(0, 0)
    load_idx(1, 1)
    load_idx(2, 2)

    zeros16 = jnp.zeros((16,), jnp.float32)

    @pl.loop(0, ZR)
    def _zero_rows(e):
        for f in range(D // 16):
            zbuf[e, pl.ds(f * 16, 16)] = zeros16

    for r0 in range(0, WB_ROWS, ZR):
        pltpu.async_copy(zbuf, acc.at[pl.ds(s * WB_ROWS + r0, ZR)], zsem)

    @pl.when(s == NS - 1)
    def _zero_tail():
        pltpu.async_copy(zbuf.at[pl.ds(0, WB_TAIL)],
                         acc.at[pl.ds(NS * WB_ROWS, WB_TAIL)], zsem)

    for r0 in range(0, WB_ROWS, ZR):
        pltpu.make_async_copy(zbuf, acc.at[pl.ds(s * WB_ROWS + r0, ZR)],
                              zsem).wait()

    @pl.when(s == NS - 1)
    def _zero_tail_wait():
        pltpu.make_async_copy(zbuf.at[pl.ds(0, WB_TAIL)],
                              acc.at[pl.ds(NS * WB_ROWS, WB_TAIL)],
                              zsem).wait()

    plsc.subcore_barrier()

    wait_idx(0, 0)
    extract_idx(0, 0)
    start_gather(0, 0)
    wait_idx(1, 1)
    extract_idx(1, 1)
    start_gather(1, 1)

    process(0, 0, 0, True, True, False)
    process(1, 1, 1, True, True, False)

    @pl.loop(2, 2 + 4 * ((N_CHUNKS - 5) // 4), step=4)
    def _main(k):
        for b in range(4):
            process(k + b, (2 + b) % NSLOT, b % NRSLOT, True, True, True)

    m0 = 2 + 4 * ((N_CHUNKS - 5) // 4)
    process(m0 + 0, (m0 + 0) % NSLOT, (m0 + 0) % NRSLOT, True, False, True)
    process(m0 + 1, (m0 + 1) % NSLOT, (m0 + 1) % NRSLOT, False, False, True)
    process(m0 + 2, (m0 + 2) % NSLOT, (m0 + 2) % NRSLOT, False, False, True)

    wait_scatter((m0 + 1) % NSLOT, (m0 + 1) % NRSLOT)
    wait_scatter((m0 + 2) % NSLOT, (m0 + 2) % NRSLOT)

    plsc.subcore_barrier()

    pltpu.sync_copy(acc.at[pl.ds(s * WB_ROWS, WB_ROWS)],
                    out.at[c, pl.ds(s * WB_ROWS, WB_ROWS)])

    @pl.when(s == NS - 1)
    def _tail():
        pltpu.sync_copy(acc.at[pl.ds(NS * WB_ROWS, WB_TAIL)],
                        out.at[c, pl.ds(NS * WB_ROWS, WB_TAIL)])


def _sc_aggregate(support, edge_index, ew):
    mesh = plsc.VectorSubcoreMesh(core_axis_name="c", subcore_axis_name="s")
    f = pl.kernel(
        _sc_body,
        out_type=jax.ShapeDtypeStruct((NC, N_NODES, D), jnp.float32),
        mesh=mesh,
        compiler_params=pltpu.CompilerParams(needs_layout_passes=False,
                                             use_tc_tiling_on_sc=False),
        scratch_types=(
            [pltpu.VMEM_SHARED((N_NODES, D), jnp.float32)]
            + [pltpu.VMEM((2, 2 * CHUNK_W), jnp.int32) for _ in range(NSLOT)]
            + [pltpu.VMEM((CHUNK,), jnp.int32) for _ in range(2 * NSLOT)]
            + [pltpu.VMEM((CHUNK,), jnp.float32) for _ in range(NSLOT)]
            + [pltpu.VMEM((CHUNK, D // 2), jnp.int32) for _ in range(NSLOT)]
            + [pltpu.VMEM((CHUNK, D), jnp.float32) for _ in range(NRSLOT)]
            + [pltpu.VMEM((ZR, D), jnp.float32)]
            + [pltpu.SemaphoreType.DMA for _ in range(3 * NSLOT + 1)]
        ),
    )
    return f(support, edge_index, ew)


@jax.jit
def kernel(x, edge_index, edge_weight, W, b, prelu_a):
    support = _tc_matmul(x, W)
    support_i32 = lax.bitcast_convert_type(
        support.reshape(N_NODES, D // 2, 2), jnp.int32)
    parts = _sc_aggregate(support_i32, edge_index, edge_weight)
    return _tc_finish(parts, b, jnp.asarray(prelu_a, jnp.float32))


# speedup vs baseline: 3.3847x; 3.3847x over previous
import functools

import jax
import jax.numpy as jnp
from jax import lax
from jax.experimental import pallas as pl
from jax.experimental.pallas import tpu as pltpu
from jax.experimental.pallas import tpu_sc as plsc

N_NODES = 10000
N_EDGES = 320000
D = 128

NC = 2
NS = 16
NW = NC * NS

E_PER_W = N_EDGES // NW
CHUNK = 80
CHUNK_W = 128
N_CHUNKS = E_PER_W // CHUNK
NSLOT = 4

ZR = 24
WB_ROWS = 624
WB_TAIL = N_NODES - NS * WB_ROWS


def _mm_body(x_ref, w_ref, o_ref):
    o_ref[...] = jnp.dot(x_ref[...], w_ref[...],
                         preferred_element_type=jnp.float32)


def _tc_matmul(x, W):
    return pl.pallas_call(
        _mm_body,
        grid=(5,),
        in_specs=[
            pl.BlockSpec((2000, D), lambda i: (i, 0)),
            pl.BlockSpec((D, D), lambda i: (0, 0)),
        ],
        out_specs=pl.BlockSpec((2000, D), lambda i: (i, 0)),
        out_shape=jax.ShapeDtypeStruct((N_NODES, D), jnp.float32),
    )(x, W)


def _fin_body(p_ref, b_ref, a_ref, o_ref):
    y = p_ref[0] + p_ref[1] + b_ref[...]
    a = a_ref[0]
    o_ref[...] = jnp.where(y >= 0, y, a * y)


def _tc_finish(parts, b, prelu_a):
    return pl.pallas_call(
        _fin_body,
        grid=(5,),
        in_specs=[
            pl.BlockSpec((2, 2000, D), lambda i: (0, i, 0)),
            pl.BlockSpec((1, D), lambda i: (0, 0)),
            pl.BlockSpec(memory_space=pltpu.SMEM),
        ],
        out_specs=pl.BlockSpec((2000, D), lambda i: (i, 0)),
        out_shape=jax.ShapeDtypeStruct((N_NODES, D), jnp.float32),
    )(parts, b.reshape(1, D), prelu_a.reshape(1))


def _sc_body(support, edge, ew, out, acc,
             eb0, eb1, eb2, eb3, sb0, sb1, sb2, sb3, db0, db1, db2, db3,
             wb0, wb1, wb2, wb3, rw0, rw1, rw2, rw3, zbuf,
             p0, p1, p2, p3, g0, g1, g2, g3, s0, s1, s2, s3, zsem):
    eb = [eb0, eb1, eb2, eb3]
    sb = [sb0, sb1, sb2, sb3]
    db = [db0, db1, db2, db3]
    wb = [wb0, wb1, wb2, wb3]
    rw = [rw0, rw1, rw2, rw3]
    psem = [p0, p1, p2, p3]
    gsem = [g0, g1, g2, g3]
    ssem = [s0, s1, s2, s3]

    c = lax.axis_index("c")
    s = lax.axis_index("s")
    wid = s * NC + c

    base0 = wid * E_PER_W

    def _slices(m):
        base = base0 + m * CHUNK
        al = pl.multiple_of(jnp.minimum((base // 128) * 128,
                                        N_EDGES - 2 * CHUNK_W), 128)
        off = pl.multiple_of(base - al, 8)
        return (edge.at[pl.ds(0, 2), pl.ds(al, 2 * CHUNK_W)],
                ew.at[pl.ds(pl.multiple_of(base, 8), CHUNK)],
                off)

    def load_idx(m, sl):
        sedge, sew, _ = _slices(m)
        pltpu.async_copy(sedge, eb[sl], psem[sl])
        pltpu.async_copy(sew, wb[sl], psem[sl])

    def wait_idx(m, sl):
        sedge, sew, _ = _slices(m)
        pltpu.make_async_copy(sedge, eb[sl], psem[sl]).wait()
        pltpu.make_async_copy(sew, wb[sl], psem[sl]).wait()

    def extract_idx(m, sl):
        _, _, off = _slices(m)
        for j in range(CHUNK // 16):
            jsl = pl.ds(j * 16, 16)
            sb[sl][jsl] = eb[sl][1, pl.ds(off + j * 16, 16)]
            db[sl][jsl] = eb[sl][0, pl.ds(off + j * 16, 16)]

    def start_gather(m, sl):
        pltpu.async_copy(support.at[sb[sl]], rw[sl], gsem[sl])

    def wait_gather(m, sl):
        pltpu.make_async_copy(support.at[sb[sl]], rw[sl], gsem[sl]).wait()

    def start_scatter(m, sl):
        pltpu.async_copy(rw[sl], acc.at[db[sl]], ssem[sl], add=True)

    def wait_scatter(m, sl):
        pltpu.make_async_copy(rw[sl], acc.at[db[sl]], ssem[sl]).wait()

    def multiply(sl):
        @pl.loop(0, CHUNK // 16)
        def _grp(g):
            wv = wb[sl][pl.ds(g * 16, 16)]
            for j in range(16):
                w = jnp.full((16,), wv[j], jnp.float32)
                e = g * 16 + j
                for f in range(D // 16):
                    fsl = pl.ds(f * 16, 16)
                    rw[sl][e, fsl] = rw[sl][e, fsl] * w

    def process(m, sl, prep, load, wait_sc):
        wait_gather(m, sl)
        multiply(sl)
        start_scatter(m, sl)
        if wait_sc:
            wait_scatter(m - 1, (sl + 3) % NSLOT)
        if prep:
            m3 = m + 3
            sl3 = (sl + 3) % NSLOT
            wait_idx(m3, sl3)
            extract_idx(m3, sl3)
            start_gather(m3, sl3)
        if load:
            load_idx(m + 4, sl)

    load_idx(0, 0)
    load_idx(1, 1)
    load_idx(2, 2)
    load_idx(3, 3)

    zeros16 = jnp.zeros((16,), jnp.float32)

    @pl.loop(0, ZR)
    def _zero_rows(e):
        for f in range(D // 16):
            zbuf[e, pl.ds(f * 16, 16)] = zeros16

    for r0 in range(0, WB_ROWS, ZR):
        pltpu.async_copy(zbuf, acc.at[pl.ds(s * WB_ROWS + r0, ZR)], zsem)

    @pl.when(s == NS - 1)
    def _zero_tail():
        pltpu.async_copy(zbuf.at[pl.ds(0, WB_TAIL)],
                         acc.at[pl.ds(NS * WB_ROWS, WB_TAIL)], zsem)

    for r0 in range(0, WB_ROWS, ZR):
        pltpu.make_async_copy(zbuf, acc.at[pl.ds(s * WB_ROWS + r0, ZR)],
                              zsem).wait()

    @pl.when(s == NS - 1)
    def _zero_tail_wait():
        pltpu.make_async_copy(zbuf.at[pl.ds(0, WB_TAIL)],
                              acc.at[pl.ds(NS * WB_ROWS, WB_TAIL)],
                              zsem).wait()

    plsc.subcore_barrier()

    for mm in range(3):
        wait_idx(mm, mm)
        extract_idx(mm, mm)
        start_gather(mm, mm)

    process(0, 0, True, True, False)

    @pl.loop(1, 1 + 4 * ((N_CHUNKS - 5) // 4), step=4)
    def _main(k):
        for b in range(4):
            process(k + b, (1 + b) % NSLOT, True, True, True)

    m0 = 1 + 4 * ((N_CHUNKS - 5) // 4)
    process(m0 + 0, (m0 + 0) % NSLOT, True, False, True)
    process(m0 + 1, (m0 + 1) % NSLOT, False, False, True)
    process(m0 + 2, (m0 + 2) % NSLOT, False, False, True)
    process(m0 + 3, (m0 + 3) % NSLOT, False, False, True)

    wait_scatter(m0 + 3, (m0 + 3) % NSLOT)

    plsc.subcore_barrier()

    pltpu.sync_copy(acc.at[pl.ds(s * WB_ROWS, WB_ROWS)],
                    out.at[c, pl.ds(s * WB_ROWS, WB_ROWS)])

    @pl.when(s == NS - 1)
    def _tail():
        pltpu.sync_copy(acc.at[pl.ds(NS * WB_ROWS, WB_TAIL)],
                        out.at[c, pl.ds(NS * WB_ROWS, WB_TAIL)])


def _sc_aggregate(support, edge_index, ew):
    mesh = plsc.VectorSubcoreMesh(core_axis_name="c", subcore_axis_name="s")
    f = pl.kernel(
        _sc_body,
        out_type=jax.ShapeDtypeStruct((NC, N_NODES, D), jnp.float32),
        mesh=mesh,
        scratch_types=(
            [pltpu.VMEM_SHARED((N_NODES, D), jnp.float32)]
            + [pltpu.VMEM((2, 2 * CHUNK_W), jnp.int32) for _ in range(NSLOT)]
            + [pltpu.VMEM((CHUNK,), jnp.int32) for _ in range(2 * NSLOT)]
            + [pltpu.VMEM((CHUNK,), jnp.float32) for _ in range(NSLOT)]
            + [pltpu.VMEM((CHUNK, D), jnp.float32) for _ in range(NSLOT)]
            + [pltpu.VMEM((ZR, D), jnp.float32)]
            + [pltpu.SemaphoreType.DMA for _ in range(3 * NSLOT + 1)]
        ),
    )
    return f(support, edge_index, ew)


@jax.jit
def kernel(x, edge_index, edge_weight, W, b, prelu_a):
    support = _tc_matmul(x, W)
    parts = _sc_aggregate(support, edge_index, edge_weight)
    return _tc_finish(parts, b, jnp.asarray(prelu_a, jnp.float32))


# speedup vs baseline: 3.4676x; 1.0245x over previous
import functools

import jax
import jax.numpy as jnp
from jax import lax
from jax.experimental import pallas as pl
from jax.experimental.pallas import tpu as pltpu
from jax.experimental.pallas import tpu_sc as plsc

N_NODES = 10000
N_EDGES = 320000
D = 128

NC = 2
NS = 16
NW = NC * NS

E_PER_W = N_EDGES // NW
CHUNK = 80
CHUNK_W = 128
N_CHUNKS = E_PER_W // CHUNK
NSLOT = 4

ZR = 24
WB_ROWS = 624
WB_TAIL = N_NODES - NS * WB_ROWS


def _mm_body(x_ref, w_ref, o_ref):
    o_ref[...] = jnp.dot(x_ref[...], w_ref[...],
                         preferred_element_type=jnp.float32)


def _tc_matmul(x, W):
    return pl.pallas_call(
        _mm_body,
        grid=(2,),
        in_specs=[
            pl.BlockSpec((5000, D), lambda i: (i, 0)),
            pl.BlockSpec((D, D), lambda i: (0, 0)),
        ],
        out_specs=pl.BlockSpec((5000, D), lambda i: (i, 0)),
        out_shape=jax.ShapeDtypeStruct((N_NODES, D), jnp.float32),
    )(x, W)


def _fin_body(p_ref, b_ref, a_ref, o_ref):
    y = p_ref[0] + p_ref[1] + b_ref[...]
    a = a_ref[0]
    o_ref[...] = jnp.where(y >= 0, y, a * y)


def _tc_finish(parts, b, prelu_a):
    return pl.pallas_call(
        _fin_body,
        grid=(2,),
        in_specs=[
            pl.BlockSpec((2, 5000, D), lambda i: (0, i, 0)),
            pl.BlockSpec((1, D), lambda i: (0, 0)),
            pl.BlockSpec(memory_space=pltpu.SMEM),
        ],
        out_specs=pl.BlockSpec((5000, D), lambda i: (i, 0)),
        out_shape=jax.ShapeDtypeStruct((N_NODES, D), jnp.float32),
    )(parts, b.reshape(1, D), prelu_a.reshape(1))


def _sc_body(support, edge, ew, out, acc,
             eb0, eb1, eb2, eb3, sb0, sb1, sb2, sb3, db0, db1, db2, db3,
             wb0, wb1, wb2, wb3, rw0, rw1, rw2, rw3, zbuf,
             p0, p1, p2, p3, g0, g1, g2, g3, s0, s1, s2, s3, zsem):
    eb = [eb0, eb1, eb2, eb3]
    sb = [sb0, sb1, sb2, sb3]
    db = [db0, db1, db2, db3]
    wb = [wb0, wb1, wb2, wb3]
    rw = [rw0, rw1, rw2, rw3]
    psem = [p0, p1, p2, p3]
    gsem = [g0, g1, g2, g3]
    ssem = [s0, s1, s2, s3]

    c = lax.axis_index("c")
    s = lax.axis_index("s")
    wid = s * NC + c

    base0 = wid * E_PER_W

    def _slices(m):
        base = base0 + m * CHUNK
        al = pl.multiple_of(jnp.minimum((base // 128) * 128,
                                        N_EDGES - 2 * CHUNK_W), 128)
        off = pl.multiple_of(base - al, 8)
        return (edge.at[pl.ds(0, 2), pl.ds(al, 2 * CHUNK_W)],
                ew.at[pl.ds(pl.multiple_of(base, 8), CHUNK)],
                off)

    def load_idx(m, sl):
        sedge, sew, _ = _slices(m)
        pltpu.async_copy(sedge, eb[sl], psem[sl])
        pltpu.async_copy(sew, wb[sl], psem[sl])

    def wait_idx(m, sl):
        sedge, sew, _ = _slices(m)
        pltpu.make_async_copy(sedge, eb[sl], psem[sl]).wait()
        pltpu.make_async_copy(sew, wb[sl], psem[sl]).wait()

    def extract_idx(m, sl):
        _, _, off = _slices(m)
        for j in range(CHUNK // 16):
            jsl = pl.ds(j * 16, 16)
            sb[sl][jsl] = eb[sl][1, pl.ds(off + j * 16, 16)]
            db[sl][jsl] = eb[sl][0, pl.ds(off + j * 16, 16)]

    def start_gather(m, sl):
        pltpu.async_copy(support.at[sb[sl]], rw[sl], gsem[sl])

    def wait_gather(m, sl):
        pltpu.make_async_copy(support.at[sb[sl]], rw[sl], gsem[sl]).wait()

    def start_scatter(m, sl):
        pltpu.async_copy(rw[sl], acc.at[db[sl]], ssem[sl], add=True)

    def wait_scatter(m, sl):
        pltpu.make_async_copy(rw[sl], acc.at[db[sl]], ssem[sl]).wait()

    def multiply(sl):
        @pl.loop(0, CHUNK // 16)
        def _grp(g):
            wv = wb[sl][pl.ds(g * 16, 16)]
            for j in range(16):
                w = jnp.full((16,), wv[j], jnp.float32)
                e = g * 16 + j
                for f in range(D // 16):
                    fsl = pl.ds(f * 16, 16)
                    rw[sl][e, fsl] = rw[sl][e, fsl] * w

    def process(m, sl, prep, load, wait_sc):
        wait_gather(m, sl)
        multiply(sl)
        start_scatter(m, sl)
        if wait_sc:
            wait_scatter(m - 1, (sl + 3) % NSLOT)
        if prep:
            m3 = m + 3
            sl3 = (sl + 3) % NSLOT
            wait_idx(m3, sl3)
            extract_idx(m3, sl3)
            start_gather(m3, sl3)
        if load:
            load_idx(m + 4, sl)

    load_idx(0, 0)
    load_idx(1, 1)
    load_idx(2, 2)
    load_idx(3, 3)

    zeros16 = jnp.zeros((16,), jnp.float32)

    @pl.loop(0, ZR)
    def _zero_rows(e):
        for f in range(D // 16):
            zbuf[e, pl.ds(f * 16, 16)] = zeros16

    for r0 in range(0, WB_ROWS, ZR):
        pltpu.async_copy(zbuf, acc.at[pl.ds(s * WB_ROWS + r0, ZR)], zsem)

    @pl.when(s == NS - 1)
    def _zero_tail():
        pltpu.async_copy(zbuf.at[pl.ds(0, WB_TAIL)],
                         acc.at[pl.ds(NS * WB_ROWS, WB_TAIL)], zsem)

    for r0 in range(0, WB_ROWS, ZR):
        pltpu.make_async_copy(zbuf, acc.at[pl.ds(s * WB_ROWS + r0, ZR)],
                              zsem).wait()

    @pl.when(s == NS - 1)
    def _zero_tail_wait():
        pltpu.make_async_copy(zbuf.at[pl.ds(0, WB_TAIL)],
                              acc.at[pl.ds(NS * WB_ROWS, WB_TAIL)],
                              zsem).wait()

    plsc.subcore_barrier()

    for mm in range(3):
        wait_idx(mm, mm)
        extract_idx(mm, mm)
        start_gather(mm, mm)

    process(0, 0, True, True, False)

    @pl.loop(1, 1 + 4 * ((N_CHUNKS - 5) // 4), step=4)
    def _main(k):
        for b in range(4):
            process(k + b, (1 + b) % NSLOT, True, True, True)

    m0 = 1 + 4 * ((N_CHUNKS - 5) // 4)
    process(m0 + 0, (m0 + 0) % NSLOT, True, False, True)
    process(m0 + 1, (m0 + 1) % NSLOT, False, False, True)
    process(m0 + 2, (m0 + 2) % NSLOT, False, False, True)
    process(m0 + 3, (m0 + 3) % NSLOT, False, False, True)

    wait_scatter(m0 + 3, (m0 + 3) % NSLOT)

    plsc.subcore_barrier()

    pltpu.sync_copy(acc.at[pl.ds(s * WB_ROWS, WB_ROWS)],
                    out.at[c, pl.ds(s * WB_ROWS, WB_ROWS)])

    @pl.when(s == NS - 1)
    def _tail():
        pltpu.sync_copy(acc.at[pl.ds(NS * WB_ROWS, WB_TAIL)],
                        out.at[c, pl.ds(NS * WB_ROWS, WB_TAIL)])


def _sc_aggregate(support, edge_index, ew):
    mesh = plsc.VectorSubcoreMesh(core_axis_name="c", subcore_axis_name="s")
    f = pl.kernel(
        _sc_body,
        out_type=jax.ShapeDtypeStruct((NC, N_NODES, D), jnp.float32),
        mesh=mesh,
        scratch_types=(
            [pltpu.VMEM_SHARED((N_NODES, D), jnp.float32)]
            + [pltpu.VMEM((2, 2 * CHUNK_W), jnp.int32) for _ in range(NSLOT)]
            + [pltpu.VMEM((CHUNK,), jnp.int32) for _ in range(2 * NSLOT)]
            + [pltpu.VMEM((CHUNK,), jnp.float32) for _ in range(NSLOT)]
            + [pltpu.VMEM((CHUNK, D), jnp.float32) for _ in range(NSLOT)]
            + [pltpu.VMEM((ZR, D), jnp.float32)]
            + [pltpu.SemaphoreType.DMA for _ in range(3 * NSLOT + 1)]
        ),
    )
    return f(support, edge_index, ew)


@jax.jit
def kernel(x, edge_index, edge_weight, W, b, prelu_a):
    support = _tc_matmul(x, W)
    parts = _sc_aggregate(support, edge_index, edge_weight)
    return _tc_finish(parts, b, jnp.asarray(prelu_a, jnp.float32))
